# Initial kernel scaffold; baseline (speedup 1.0000x reference)
import jax
import jax.numpy as jnp
from jax.experimental import pallas as pl


def kernel(x, edge_index, precomp_neigh_edge, precomp_self_edge, connection, conv1_W_ang, conv1_W_ch, conv1_W_self_ang, conv1_W_self_ch, conv1_Wq, conv1_Wk, conv1_bias, conv2_W_ang, conv2_W_ch, conv2_W_self_ang, conv2_W_self_ch, conv2_Wq, conv2_Wk, conv2_bias):
    raise NotImplementedError("write your pallas kernel here")


# speedup vs baseline: 8.9443x; 8.9443x over previous
import functools
import numpy as np
import jax
import jax.numpy as jnp
from jax import lax
from jax.experimental import pallas as pl
from jax.experimental.pallas import tpu as pltpu
from jax.experimental.pallas import tpu_sc as plsc

N = 10000
NPAD = 10240
E = 160000
C = 128
D = 5
NH = 2
F = C * D
FX = F + C
CB = 128
NBLK = FX // CB
NCORES = 2
NSUB = 16
NW = NCORES * NSUB

BE = 2000
BN = 2048

XCH = 40
QCH = 200
ECH = 200

_f32 = jnp.float32


def _nonlin_mats(order=2, num_samples=5):
    thetas = 2.0 * np.pi * np.arange(num_samples) / num_samples
    cols = [np.ones(num_samples)]
    for m in range(1, order + 1):
        cols.append(np.cos(m * thetas))
        cols.append(np.sin(m * thetas))
    B = np.stack(cols, axis=1)
    scale = np.array([1.0] + [2.0] * (2 * order))
    Binv = (B * scale[None, :]) / num_samples
    return B.astype(np.float32), Binv.astype(np.float32)


_BMAT, _BINV = _nonlin_mats()


def _prep_body(x_ref, wch_ref, wq_ref, xw_ref, qn_ref):
    wch = wch_ref[...]
    for d in range(D):
        xw_ref[:, d * C:(d + 1) * C] = jnp.dot(
            x_ref[:, d * C:(d + 1) * C], wch, preferred_element_type=_f32)
    qn_ref[...] = jnp.dot(x_ref[:, 0:C], wq_ref[...], preferred_element_type=_f32)


def _prep(x2, w_ch, wq):
    return pl.pallas_call(
        _prep_body,
        grid=(NPAD // BN,),
        in_specs=[
            pl.BlockSpec((BN, F), lambda i: (i, 0)),
            pl.BlockSpec((C, C), lambda i: (0, 0)),
            pl.BlockSpec((C, C), lambda i: (0, 0)),
        ],
        out_specs=[
            pl.BlockSpec((BN, F), lambda i: (i, 0)),
            pl.BlockSpec((BN, C), lambda i: (i, 0)),
        ],
        out_shape=[
            jax.ShapeDtypeStruct((NPAD, F), _f32),
            jax.ShapeDtypeStruct((NPAD, C), _f32),
        ],
    )(x2, w_ch, wq)


def _gather_body(src_hbm, dst_hbm, xw_hbm, qn_hbm, xs_out, qd_out,
                 xibuf, xbuf, qibuf, qbuf, sem):
    cid = lax.axis_index("c")
    sid = lax.axis_index("s")
    wid = sid * NCORES + cid
    ept = E // NW
    base = wid * ept

    def xw_step(i, carry):
        e0 = base + i * XCH
        pltpu.sync_copy(src_hbm.at[pl.ds(e0, XCH)], xibuf)
        pltpu.async_copy(xw_hbm.at[xibuf], xbuf, sem).wait()
        pltpu.sync_copy(xbuf, xs_out.at[pl.ds(e0, XCH)])
        return carry

    lax.fori_loop(0, ept // XCH, xw_step, 0)

    def qn_step(i, carry):
        e0 = base + i * QCH
        pltpu.sync_copy(dst_hbm.at[pl.ds(e0, QCH)], qibuf)
        pltpu.async_copy(qn_hbm.at[qibuf], qbuf, sem).wait()
        pltpu.sync_copy(qbuf, qd_out.at[pl.ds(e0, QCH)])
        return carry

    lax.fori_loop(0, ept // QCH, qn_step, 0)


def _gather(src, dst, xw2, qn):
    mesh = plsc.VectorSubcoreMesh(core_axis_name="c", subcore_axis_name="s")
    f = pl.kernel(
        _gather_body,
        out_type=[
            jax.ShapeDtypeStruct((E, F), _f32),
            jax.ShapeDtypeStruct((E, C), _f32),
        ],
        mesh=mesh,
        scratch_types=[
            pltpu.VMEM((XCH,), jnp.int32),
            pltpu.VMEM((XCH, F), _f32),
            pltpu.VMEM((QCH,), jnp.int32),
            pltpu.VMEM((QCH, C), _f32),
            pltpu.SemaphoreType.DMA,
        ],
    )
    return f(src, dst, xw2, qn)


def _edge_body(xs_ref, qd_ref, pne_ref, conn_ref, wam_ref, wk_ref,
               mh_ref, mht_ref, out_ref):
    kern = jnp.dot(pne_ref[...], wam_ref[...], preferred_element_type=_f32)
    th = conn_ref[...]
    c1 = jnp.cos(th)
    s1 = jnp.sin(th)
    c2 = jnp.cos(2.0 * th)
    s2 = jnp.sin(2.0 * th)
    k0 = kern[:, 0:5]
    kc1 = kern[:, 5:10]
    ks1 = kern[:, 10:15]
    kc2 = kern[:, 15:20]
    ks2 = kern[:, 20:25]
    cs = [
        k0,
        c1 * kc1 + s1 * ks1,
        -s1 * kc1 + c1 * ks1,
        c2 * kc2 + s2 * ks2,
        -s2 * kc2 + c2 * ks2,
    ]
    xs = [xs_ref[:, d * C:(d + 1) * C] for d in range(D)]
    msg = []
    for k in range(D):
        acc = cs[0][:, k:k + 1] * xs[0]
        for d in range(1, D):
            acc = acc + cs[d][:, k:k + 1] * xs[d]
        msg.append(acc)
    katt = jnp.dot(msg[0], wk_ref[...], preferred_element_type=_f32)
    qk = qd_ref[...] * katt
    lg = jnp.dot(qk, mh_ref[...], preferred_element_type=_f32) * (1.0 / 8.0)
    ex = jnp.exp(lg)
    exb = jnp.dot(ex, mht_ref[...], preferred_element_type=_f32)
    for k in range(D):
        out_ref[:, k * C:(k + 1) * C] = msg[k] * exb
    out_ref[:, F:FX] = exb


def _edge(xs2, qd, pne_f, conn2, wam, wk, mh, mht):
    return pl.pallas_call(
        _edge_body,
        grid=(E // BE,),
        in_specs=[
            pl.BlockSpec((BE, F), lambda i: (i, 0)),
            pl.BlockSpec((BE, C), lambda i: (i, 0)),
            pl.BlockSpec((BE, 10), lambda i: (i, 0)),
            pl.BlockSpec((BE, 1), lambda i: (i, 0)),
            pl.BlockSpec((10, 25), lambda i: (0, 0)),
            pl.BlockSpec((C, C), lambda i: (0, 0)),
            pl.BlockSpec((C, NH), lambda i: (0, 0)),
            pl.BlockSpec((NH, C), lambda i: (0, 0)),
        ],
        out_specs=pl.BlockSpec((BE, FX), lambda i: (i, 0)),
        out_shape=jax.ShapeDtypeStruct((E, FX), _f32),
    )(xs2, qd, pne_f, conn2, wam, wk, mh, mht)


def _scatter_body(exmx_hbm, dst_hbm, zeros_hbm, agg_out, ebuf, ibuf, acc):
    cid = lax.axis_index("c")
    sid = lax.axis_index("s")
    ept = E // NSUB
    rows_per_tile = NPAD // NSUB
    for p in range(NBLK // NCORES):
        blk = cid * (NBLK // NCORES) + p
        col0 = blk * CB
        for r in range(rows_per_tile // 128):
            r0 = sid * rows_per_tile + r * 128
            pltpu.sync_copy(zeros_hbm, acc.at[pl.ds(r0, 128)])
        plsc.subcore_barrier()

        def step(j, carry):
            e0 = sid * ept + j * ECH
            pltpu.sync_copy(
                exmx_hbm.at[pl.ds(e0, ECH), pl.ds(col0, CB)], ebuf)
            pltpu.sync_copy(dst_hbm.at[pl.ds(e0, ECH)], ibuf)
            pltpu.sync_copy(ebuf, acc.at[ibuf], add=True)
            return carry

        lax.fori_loop(0, ept // ECH, step, 0)
        plsc.subcore_barrier()
        for r in range(rows_per_tile // 128):
            r0 = sid * rows_per_tile + r * 128
            pltpu.sync_copy(acc.at[pl.ds(r0, 128)], ebuf.at[pl.ds(0, 128)])
            pltpu.sync_copy(ebuf.at[pl.ds(0, 128)],
                            agg_out.at[pl.ds(r0, 128), pl.ds(col0, CB)])
        plsc.subcore_barrier()


def _scatter(exmx, dst, zeros_cb):
    mesh = plsc.VectorSubcoreMesh(core_axis_name="c", subcore_axis_name="s")
    f = pl.kernel(
        _scatter_body,
        out_type=jax.ShapeDtypeStruct((NPAD, FX), _f32),
        mesh=mesh,
        scratch_types=[
            pltpu.VMEM((ECH, CB), _f32),
            pltpu.VMEM((ECH,), jnp.int32),
            pltpu.VMEM_SHARED((NPAD, CB), _f32),
        ],
    )
    return f(exmx, dst, zeros_cb)


def _node_body(agg_ref, x_ref, res_ref, pse_ref, wsam_ref, wsch_ref, bias_ref,
               out_ref, *, residual):
    kern_s = jnp.dot(pse_ref[...], wsam_ref[...], preferred_element_type=_f32)
    wsch = wsch_ref[...]
    xws = [jnp.dot(x_ref[:, d * C:(d + 1) * C], wsch,
                   preferred_element_type=_f32) for d in range(D)]
    den = agg_ref[:, F:FX]
    r = 1.0 / (den + 1e-9)
    ys = []
    for k in range(D):
        sm = kern_s[:, k:k + 1] * xws[0]
        for d in range(1, D):
            sm = sm + kern_s[:, d * D + k:d * D + k + 1] * xws[d]
        y = agg_ref[:, k * C:(k + 1) * C] * r + sm
        if k == 0:
            y = y + bias_ref[...]
        if residual:
            y = y + res_ref[:, k * C:(k + 1) * C]
        ys.append(y)
    sps = []
    for s in range(D):
        sp = float(_BMAT[s, 0]) * ys[0]
        for k in range(1, D):
            sp = sp + float(_BMAT[s, k]) * ys[k]
        sps.append(jnp.maximum(sp, 0.0))
    for d in range(D):
        z = float(_BINV[0, d]) * sps[0]
        for s in range(1, D):
            z = z + float(_BINV[s, d]) * sps[s]
        out_ref[:, d * C:(d + 1) * C] = z


def _node(aggf, x2, res2, pse_f, wsam, wsch, bias2, residual):
    body = functools.partial(_node_body, residual=residual)
    return pl.pallas_call(
        body,
        grid=(NPAD // BN,),
        in_specs=[
            pl.BlockSpec((BN, FX), lambda i: (i, 0)),
            pl.BlockSpec((BN, F), lambda i: (i, 0)),
            pl.BlockSpec((BN, F), lambda i: (i, 0)),
            pl.BlockSpec((BN, 10), lambda i: (i, 0)),
            pl.BlockSpec((10, 25), lambda i: (0, 0)),
            pl.BlockSpec((C, C), lambda i: (0, 0)),
            pl.BlockSpec((1, C), lambda i: (0, 0)),
        ],
        out_specs=pl.BlockSpec((BN, F), lambda i: (i, 0)),
        out_shape=jax.ShapeDtypeStruct((NPAD, F), _f32),
    )(aggf, x2, res2, pse_f, wsam, wsch, bias2)


def _flatten_nodes(a):
    a2 = jnp.transpose(a, (0, 2, 1)).reshape(a.shape[0], -1)
    return jnp.pad(a2, ((0, NPAD - a.shape[0]), (0, 0)))


@jax.jit
def kernel(x, edge_index, precomp_neigh_edge, precomp_self_edge, connection,
           conv1_W_ang, conv1_W_ch, conv1_W_self_ang, conv1_W_self_ch,
           conv1_Wq, conv1_Wk, conv1_bias,
           conv2_W_ang, conv2_W_ch, conv2_W_self_ang, conv2_W_self_ch,
           conv2_Wq, conv2_Wk, conv2_bias):
    x2 = _flatten_nodes(x)
    dst = edge_index[:, 0]
    src = edge_index[:, 1]
    pne_f = precomp_neigh_edge.reshape(E, -1)
    pse_f = jnp.pad(precomp_self_edge.reshape(N, -1), ((0, NPAD - N), (0, 0)))
    conn2 = connection.reshape(E, 1)
    heads = (jnp.arange(C) // (C // NH))[:, None] == jnp.arange(NH)[None, :]
    mh = heads.astype(_f32)
    mht = mh.T
    zeros_cb = jnp.zeros((128, CB), _f32)

    y = x2
    for wang, wch, wsang, wsch, wq, wk, bias, residual in (
        (conv1_W_ang, conv1_W_ch, conv1_W_self_ang, conv1_W_self_ch,
         conv1_Wq, conv1_Wk, conv1_bias, False),
        (conv2_W_ang, conv2_W_ch, conv2_W_self_ang, conv2_W_self_ch,
         conv2_Wq, conv2_Wk, conv2_bias, True),
    ):
        wam = jnp.transpose(wang, (1, 0, 2, 3)).reshape(10, 25)
        wsam = jnp.transpose(wsang, (1, 0, 2, 3)).reshape(10, 25)
        bias2 = bias.reshape(1, C)
        xw2, qn = _prep(y, wch, wq)
        xs2, qd = _gather(src, dst, xw2, qn)
        exmx = _edge(xs2, qd, pne_f, conn2, wam, wk, mh, mht)
        aggf = _scatter(exmx, dst, zeros_cb)
        y = _node(aggf, y, x2, pse_f, wsam, wsch, bias2, residual)

    out = y[:N].reshape(N, D, C)
    return jnp.transpose(out, (0, 2, 1))


# speedup vs baseline: 12.2497x; 1.3696x over previous
import functools
import numpy as np
import jax
import jax.numpy as jnp
from jax import lax
from jax.experimental import pallas as pl
from jax.experimental.pallas import tpu as pltpu
from jax.experimental.pallas import tpu_sc as plsc

N = 10000
NPAD = 10240
E = 160000
C = 128
D = 5
NH = 2
F = C * D
FX = F + C
CB = 128
NBLK = FX // CB
NCORES = 2
NSUB = 16
NW = NCORES * NSUB

BE = 2000
BN = 2048

XCH = 40
QCH = 200
ECH = 200

_f32 = jnp.float32


def _nonlin_mats(order=2, num_samples=5):
    thetas = 2.0 * np.pi * np.arange(num_samples) / num_samples
    cols = [np.ones(num_samples)]
    for m in range(1, order + 1):
        cols.append(np.cos(m * thetas))
        cols.append(np.sin(m * thetas))
    B = np.stack(cols, axis=1)
    scale = np.array([1.0] + [2.0] * (2 * order))
    Binv = (B * scale[None, :]) / num_samples
    return B.astype(np.float32), Binv.astype(np.float32)


_BMAT, _BINV = _nonlin_mats()


def _edge_consts():
    sw = np.zeros((25, 25), np.float32)
    for j in range(5):
        sw[10 + j, 5 + j] = 1.0
        sw[5 + j, 10 + j] = 1.0
        sw[20 + j, 15 + j] = 1.0
        sw[15 + j, 20 + j] = 1.0
    ga = np.zeros((3, 25), np.float32)
    ga[0, 0:5] = 1.0
    ga[1, 5:15] = 1.0
    ga[2, 15:25] = 1.0
    gb = np.zeros((2, 25), np.float32)
    gb[0, 5:10] = 1.0
    gb[0, 10:15] = -1.0
    gb[1, 15:20] = 1.0
    gb[1, 20:25] = -1.0
    sbc = np.zeros((25, 5 * 640), np.float32)
    for k in range(5):
        for d in range(5):
            sbc[d * 5 + k, k * 640 + d * 128:k * 640 + (d + 1) * 128] = 1.0
    return sw, ga, gb, sbc


_SW, _GA, _GB, _SBC = _edge_consts()


def _prep_body(x_ref, wch_ref, wq_ref, xw_ref, qn_ref):
    wch = wch_ref[...]
    for d in range(D):
        xw_ref[:, d * C:(d + 1) * C] = jnp.dot(
            x_ref[:, d * C:(d + 1) * C], wch, preferred_element_type=_f32)
    qn_ref[...] = jnp.dot(x_ref[:, 0:C], wq_ref[...], preferred_element_type=_f32)


def _prep(x2, w_ch, wq):
    return pl.pallas_call(
        _prep_body,
        grid=(NPAD // BN,),
        in_specs=[
            pl.BlockSpec((BN, F), lambda i: (i, 0)),
            pl.BlockSpec((C, C), lambda i: (0, 0)),
            pl.BlockSpec((C, C), lambda i: (0, 0)),
        ],
        out_specs=[
            pl.BlockSpec((BN, F), lambda i: (i, 0)),
            pl.BlockSpec((BN, C), lambda i: (i, 0)),
        ],
        out_shape=[
            jax.ShapeDtypeStruct((NPAD, F), _f32),
            jax.ShapeDtypeStruct((NPAD, C), _f32),
        ],
    )(x2, w_ch, wq)


def _gather_body(src_hbm, dst_hbm, xw_hbm, qn_hbm, xs_out, qd_out,
                 xibuf, xbuf, qibuf, qbuf, sem):
    cid = lax.axis_index("c")
    sid = lax.axis_index("s")
    wid = sid * NCORES + cid
    ept = E // NW
    base = wid * ept

    def xw_step(i, carry):
        e0 = base + i * XCH
        pltpu.sync_copy(src_hbm.at[pl.ds(e0, XCH)], xibuf)
        pltpu.async_copy(xw_hbm.at[xibuf], xbuf, sem).wait()
        pltpu.sync_copy(xbuf, xs_out.at[pl.ds(e0, XCH)])
        return carry

    lax.fori_loop(0, ept // XCH, xw_step, 0)

    def qn_step(i, carry):
        e0 = base + i * QCH
        pltpu.sync_copy(dst_hbm.at[pl.ds(e0, QCH)], qibuf)
        pltpu.async_copy(qn_hbm.at[qibuf], qbuf, sem).wait()
        pltpu.sync_copy(qbuf, qd_out.at[pl.ds(e0, QCH)])
        return carry

    lax.fori_loop(0, ept // QCH, qn_step, 0)


def _gather(src, dst, xw2, qn):
    mesh = plsc.VectorSubcoreMesh(core_axis_name="c", subcore_axis_name="s")
    f = pl.kernel(
        _gather_body,
        out_type=[
            jax.ShapeDtypeStruct((E, F), _f32),
            jax.ShapeDtypeStruct((E, C), _f32),
        ],
        mesh=mesh,
        scratch_types=[
            pltpu.VMEM((XCH,), jnp.int32),
            pltpu.VMEM((XCH, F), _f32),
            pltpu.VMEM((QCH,), jnp.int32),
            pltpu.VMEM((QCH, C), _f32),
            pltpu.SemaphoreType.DMA,
        ],
    )
    return f(src, dst, xw2, qn)


def _edge_body(xs_ref, qd_ref, pne_ref, conn_ref, wam_ref, wk_ref,
               mh_ref, mht_ref, sw_ref, ga_ref, gb_ref, sbc_ref, out_ref):
    kern = jnp.dot(pne_ref[...], wam_ref[...], preferred_element_type=_f32)
    th = conn_ref[...]
    c1 = jnp.cos(th)
    s1 = jnp.sin(th)
    c2 = c1 * c1 - s1 * s1
    s2 = 2.0 * s1 * c1
    one = jnp.ones_like(th)
    t3 = jnp.concatenate([one, c1, c2], axis=1)
    t2 = jnp.concatenate([s1, s2], axis=1)
    a = jnp.dot(t3, ga_ref[...], preferred_element_type=_f32)
    b = jnp.dot(t2, gb_ref[...], preferred_element_type=_f32)
    kern_sw = jnp.dot(kern, sw_ref[...], preferred_element_type=_f32)
    cc = a * kern + b * kern_sw
    xs = [xs_ref[:, d * C:(d + 1) * C] for d in range(D)]
    msg = []
    for k in range(D):
        cb = jnp.dot(cc, sbc_ref[:, k * F:(k + 1) * F],
                     preferred_element_type=_f32)
        acc = cb[:, 0:C] * xs[0]
        for d in range(1, D):
            acc = acc + cb[:, d * C:(d + 1) * C] * xs[d]
        msg.append(acc)
    katt = jnp.dot(msg[0], wk_ref[...], preferred_element_type=_f32)
    qk = qd_ref[...] * katt
    lg = jnp.dot(qk, mh_ref[...], preferred_element_type=_f32)
    ex = jnp.exp(lg)
    exb = jnp.dot(ex, mht_ref[...], preferred_element_type=_f32)
    for k in range(D):
        out_ref[:, k * C:(k + 1) * C] = msg[k] * exb
    out_ref[:, F:FX] = exb


def _edge(xs2, qd, pne_f, conn2, wam, wk, mh, mht, sw, ga, gb, sbc):
    return pl.pallas_call(
        _edge_body,
        grid=(E // BE,),
        in_specs=[
            pl.BlockSpec((BE, F), lambda i: (i, 0)),
            pl.BlockSpec((BE, C), lambda i: (i, 0)),
            pl.BlockSpec((BE, 10), lambda i: (i, 0)),
            pl.BlockSpec((BE, 1), lambda i: (i, 0)),
            pl.BlockSpec((10, 25), lambda i: (0, 0)),
            pl.BlockSpec((C, C), lambda i: (0, 0)),
            pl.BlockSpec((C, NH), lambda i: (0, 0)),
            pl.BlockSpec((NH, C), lambda i: (0, 0)),
            pl.BlockSpec((25, 25), lambda i: (0, 0)),
            pl.BlockSpec((3, 25), lambda i: (0, 0)),
            pl.BlockSpec((2, 25), lambda i: (0, 0)),
            pl.BlockSpec((25, D * F), lambda i: (0, 0)),
        ],
        out_specs=pl.BlockSpec((BE, FX), lambda i: (i, 0)),
        out_shape=jax.ShapeDtypeStruct((E, FX), _f32),
    )(xs2, qd, pne_f, conn2, wam, wk, mh, mht, sw, ga, gb, sbc)


def _scatter_body(exmx_hbm, dst_hbm, zeros_hbm, agg_out, ebuf, ibuf, acc):
    cid = lax.axis_index("c")
    sid = lax.axis_index("s")
    ept = E // NSUB
    rows_per_tile = NPAD // NSUB
    for p in range(NBLK // NCORES):
        blk = cid * (NBLK // NCORES) + p
        col0 = blk * CB
        for r in range(rows_per_tile // 128):
            r0 = sid * rows_per_tile + r * 128
            pltpu.sync_copy(zeros_hbm, acc.at[pl.ds(r0, 128)])
        plsc.subcore_barrier()

        def step(j, carry):
            e0 = sid * ept + j * ECH
            pltpu.sync_copy(
                exmx_hbm.at[pl.ds(e0, ECH), pl.ds(col0, CB)], ebuf)
            pltpu.sync_copy(dst_hbm.at[pl.ds(e0, ECH)], ibuf)
            pltpu.sync_copy(ebuf, acc.at[ibuf], add=True)
            return carry

        lax.fori_loop(0, ept // ECH, step, 0)
        plsc.subcore_barrier()
        for r in range(rows_per_tile // 128):
            r0 = sid * rows_per_tile + r * 128
            pltpu.sync_copy(acc.at[pl.ds(r0, 128)], ebuf.at[pl.ds(0, 128)])
            pltpu.sync_copy(ebuf.at[pl.ds(0, 128)],
                            agg_out.at[pl.ds(r0, 128), pl.ds(col0, CB)])
        plsc.subcore_barrier()


def _scatter(exmx, dst, zeros_cb):
    mesh = plsc.VectorSubcoreMesh(core_axis_name="c", subcore_axis_name="s")
    f = pl.kernel(
        _scatter_body,
        out_type=jax.ShapeDtypeStruct((NPAD, FX), _f32),
        mesh=mesh,
        scratch_types=[
            pltpu.VMEM((ECH, CB), _f32),
            pltpu.VMEM((ECH,), jnp.int32),
            pltpu.VMEM_SHARED((NPAD, CB), _f32),
        ],
    )
    return f(exmx, dst, zeros_cb)


def _node_body(agg_ref, x_ref, res_ref, pse_ref, wsam_ref, wsch_ref, bias_ref,
               out_ref, *, residual):
    kern_s = jnp.dot(pse_ref[...], wsam_ref[...], preferred_element_type=_f32)
    wsch = wsch_ref[...]
    xws = [jnp.dot(x_ref[:, d * C:(d + 1) * C], wsch,
                   preferred_element_type=_f32) for d in range(D)]
    den = agg_ref[:, F:FX]
    r = 1.0 / (den + 1e-9)
    ys = []
    for k in range(D):
        sm = kern_s[:, k:k + 1] * xws[0]
        for d in range(1, D):
            sm = sm + kern_s[:, d * D + k:d * D + k + 1] * xws[d]
        y = agg_ref[:, k * C:(k + 1) * C] * r + sm
        if k == 0:
            y = y + bias_ref[...]
        if residual:
            y = y + res_ref[:, k * C:(k + 1) * C]
        ys.append(y)
    sps = []
    for s in range(D):
        sp = float(_BMAT[s, 0]) * ys[0]
        for k in range(1, D):
            sp = sp + float(_BMAT[s, k]) * ys[k]
        sps.append(jnp.maximum(sp, 0.0))
    for d in range(D):
        z = float(_BINV[0, d]) * sps[0]
        for s in range(1, D):
            z = z + float(_BINV[s, d]) * sps[s]
        out_ref[:, d * C:(d + 1) * C] = z


def _node(aggf, x2, res2, pse_f, wsam, wsch, bias2, residual):
    body = functools.partial(_node_body, residual=residual)
    return pl.pallas_call(
        body,
        grid=(NPAD // BN,),
        in_specs=[
            pl.BlockSpec((BN, FX), lambda i: (i, 0)),
            pl.BlockSpec((BN, F), lambda i: (i, 0)),
            pl.BlockSpec((BN, F), lambda i: (i, 0)),
            pl.BlockSpec((BN, 10), lambda i: (i, 0)),
            pl.BlockSpec((10, 25), lambda i: (0, 0)),
            pl.BlockSpec((C, C), lambda i: (0, 0)),
            pl.BlockSpec((1, C), lambda i: (0, 0)),
        ],
        out_specs=pl.BlockSpec((BN, F), lambda i: (i, 0)),
        out_shape=jax.ShapeDtypeStruct((NPAD, F), _f32),
    )(aggf, x2, res2, pse_f, wsam, wsch, bias2)


def _flatten_nodes(a):
    a2 = jnp.transpose(a, (0, 2, 1)).reshape(a.shape[0], -1)
    return jnp.pad(a2, ((0, NPAD - a.shape[0]), (0, 0)))


@jax.jit
def kernel(x, edge_index, precomp_neigh_edge, precomp_self_edge, connection,
           conv1_W_ang, conv1_W_ch, conv1_W_self_ang, conv1_W_self_ch,
           conv1_Wq, conv1_Wk, conv1_bias,
           conv2_W_ang, conv2_W_ch, conv2_W_self_ang, conv2_W_self_ch,
           conv2_Wq, conv2_Wk, conv2_bias):
    x2 = _flatten_nodes(x)
    dst = edge_index[:, 0]
    src = edge_index[:, 1]
    pne_f = precomp_neigh_edge.reshape(E, -1)
    pse_f = jnp.pad(precomp_self_edge.reshape(N, -1), ((0, NPAD - N), (0, 0)))
    conn2 = connection.reshape(E, 1)
    heads = (jnp.arange(C) // (C // NH))[:, None] == jnp.arange(NH)[None, :]
    mh = heads.astype(_f32) * (1.0 / 8.0)
    mht = heads.astype(_f32).T
    zeros_cb = jnp.zeros((128, CB), _f32)
    sw = jnp.asarray(_SW)
    ga = jnp.asarray(_GA)
    gb = jnp.asarray(_GB)
    sbc = jnp.asarray(_SBC)

    y = x2
    for wang, wch, wsang, wsch, wq, wk, bias, residual in (
        (conv1_W_ang, conv1_W_ch, conv1_W_self_ang, conv1_W_self_ch,
         conv1_Wq, conv1_Wk, conv1_bias, False),
        (conv2_W_ang, conv2_W_ch, conv2_W_self_ang, conv2_W_self_ch,
         conv2_Wq, conv2_Wk, conv2_bias, True),
    ):
        wam = jnp.transpose(wang, (1, 0, 2, 3)).reshape(10, 25)
        wsam = jnp.transpose(wsang, (1, 0, 2, 3)).reshape(10, 25)
        bias2 = bias.reshape(1, C)
        xw2, qn = _prep(y, wch, wq)
        xs2, qd = _gather(src, dst, xw2, qn)
        exmx = _edge(xs2, qd, pne_f, conn2, wam, wk, mh, mht, sw, ga, gb, sbc)
        aggf = _scatter(exmx, dst, zeros_cb)
        y = _node(aggf, y, x2, pse_f, wsam, wsch, bias2, residual)

    out = y[:N].reshape(N, D, C)
    return jnp.transpose(out, (0, 2, 1))


# speedup vs baseline: 14.0739x; 1.1489x over previous
import functools
import numpy as np
import jax
import jax.numpy as jnp
from jax import lax
from jax.experimental import pallas as pl
from jax.experimental.pallas import tpu as pltpu
from jax.experimental.pallas import tpu_sc as plsc

N = 10000
NPAD = 10240
E = 160000
C = 128
D = 5
NH = 2
F = C * D
FX = F + C
CB = 128
NBLK = FX // CB
NCORES = 2
NSUB = 16
NW = NCORES * NSUB

BE = 1280
BN = 2048

XCH = 40
QCH = 200
ECH = 200

_f32 = jnp.float32


def _nonlin_mats(order=2, num_samples=5):
    thetas = 2.0 * np.pi * np.arange(num_samples) / num_samples
    cols = [np.ones(num_samples)]
    for m in range(1, order + 1):
        cols.append(np.cos(m * thetas))
        cols.append(np.sin(m * thetas))
    B = np.stack(cols, axis=1)
    scale = np.array([1.0] + [2.0] * (2 * order))
    Binv = (B * scale[None, :]) / num_samples
    return B.astype(np.float32), Binv.astype(np.float32)


_BMAT, _BINV = _nonlin_mats()


def _edge_consts():
    sw = np.zeros((25, 25), np.float32)
    for j in range(5):
        sw[10 + j, 5 + j] = 1.0
        sw[5 + j, 10 + j] = 1.0
        sw[20 + j, 15 + j] = 1.0
        sw[15 + j, 20 + j] = 1.0
    ga = np.zeros((3, 25), np.float32)
    ga[0, 0:5] = 1.0
    ga[1, 5:15] = 1.0
    ga[2, 15:25] = 1.0
    gb = np.zeros((2, 25), np.float32)
    gb[0, 5:10] = 1.0
    gb[0, 10:15] = -1.0
    gb[1, 15:20] = 1.0
    gb[1, 20:25] = -1.0
    sbc = np.zeros((25, 5 * 640), np.float32)
    for k in range(5):
        for d in range(5):
            sbc[d * 5 + k, k * 640 + d * 128:k * 640 + (d + 1) * 128] = 1.0
    return sw, ga, gb, sbc


_SW, _GA, _GB, _SBC = _edge_consts()


def _prep_body(x_ref, wch_ref, wq_ref, xw_ref, qn_ref):
    wch = wch_ref[...]
    for d in range(D):
        xw_ref[:, d * C:(d + 1) * C] = jnp.dot(
            x_ref[:, d * C:(d + 1) * C], wch, preferred_element_type=_f32)
    qn_ref[...] = jnp.dot(x_ref[:, 0:C], wq_ref[...], preferred_element_type=_f32)


def _prep(x2, w_ch, wq):
    return pl.pallas_call(
        _prep_body,
        grid=(NPAD // BN,),
        in_specs=[
            pl.BlockSpec((BN, F), lambda i: (i, 0)),
            pl.BlockSpec((C, C), lambda i: (0, 0)),
            pl.BlockSpec((C, C), lambda i: (0, 0)),
        ],
        out_specs=[
            pl.BlockSpec((BN, F), lambda i: (i, 0)),
            pl.BlockSpec((BN, C), lambda i: (i, 0)),
        ],
        out_shape=[
            jax.ShapeDtypeStruct((NPAD, F), _f32),
            jax.ShapeDtypeStruct((NPAD, C), _f32),
        ],
    )(x2, w_ch, wq)


def _gather_body(src_hbm, dst_hbm, xw_hbm, qn_hbm, xs_out, qd_out,
                 xibuf, xbuf, qibuf, qbuf, sem):
    cid = lax.axis_index("c")
    sid = lax.axis_index("s")
    wid = sid * NCORES + cid
    ept = E // NW
    base = wid * ept

    def xw_step(i, carry):
        e0 = base + i * XCH
        pltpu.sync_copy(src_hbm.at[pl.ds(e0, XCH)], xibuf)
        pltpu.async_copy(xw_hbm.at[xibuf], xbuf, sem).wait()
        pltpu.sync_copy(xbuf, xs_out.at[pl.ds(e0, XCH)])
        return carry

    lax.fori_loop(0, ept // XCH, xw_step, 0)

    def qn_step(i, carry):
        e0 = base + i * QCH
        pltpu.sync_copy(dst_hbm.at[pl.ds(e0, QCH)], qibuf)
        pltpu.async_copy(qn_hbm.at[qibuf], qbuf, sem).wait()
        pltpu.sync_copy(qbuf, qd_out.at[pl.ds(e0, QCH)])
        return carry

    lax.fori_loop(0, ept // QCH, qn_step, 0)


def _gather(src, dst, xw2, qn):
    mesh = plsc.VectorSubcoreMesh(core_axis_name="c", subcore_axis_name="s")
    f = pl.kernel(
        _gather_body,
        out_type=[
            jax.ShapeDtypeStruct((E, F), _f32),
            jax.ShapeDtypeStruct((E, C), _f32),
        ],
        mesh=mesh,
        scratch_types=[
            pltpu.VMEM((XCH,), jnp.int32),
            pltpu.VMEM((XCH, F), _f32),
            pltpu.VMEM((QCH,), jnp.int32),
            pltpu.VMEM((QCH, C), _f32),
            pltpu.SemaphoreType.DMA,
        ],
    )
    return f(src, dst, xw2, qn)


def _edge_body(xs_ref, qd_ref, pnet_ref, conn_ref, wamt_ref, wk_ref,
               mh_ref, mht_ref, sw_ref, gat_ref, gbt_ref, sbc_ref, out_ref):
    kern_t = jnp.dot(wamt_ref[...], pnet_ref[...], preferred_element_type=_f32)
    th = conn_ref[...]
    c1 = jnp.cos(th)
    s1 = jnp.sin(th)
    c2 = c1 * c1 - s1 * s1
    s2 = 2.0 * s1 * c1
    one = jnp.ones_like(th)
    t3 = jnp.concatenate([one, c1, c2], axis=0)
    t2 = jnp.concatenate([s1, s2], axis=0)
    a = jnp.dot(gat_ref[...], t3, preferred_element_type=_f32)
    b = jnp.dot(gbt_ref[...], t2, preferred_element_type=_f32)
    kern_sw = jnp.dot(sw_ref[...], kern_t, preferred_element_type=_f32)
    cc_t = a * kern_t + b * kern_sw
    xs = [xs_ref[:, d * C:(d + 1) * C] for d in range(D)]
    msg = []
    for k in range(D):
        cb = jax.lax.dot_general(
            cc_t, sbc_ref[:, k * F:(k + 1) * F],
            (((0,), (0,)), ((), ())), preferred_element_type=_f32)
        acc = cb[:, 0:C] * xs[0]
        for d in range(1, D):
            acc = acc + cb[:, d * C:(d + 1) * C] * xs[d]
        msg.append(acc)
    katt = jnp.dot(msg[0], wk_ref[...], preferred_element_type=_f32)
    qk = qd_ref[...] * katt
    lg = jnp.dot(qk, mh_ref[...], preferred_element_type=_f32)
    ex = jnp.exp(lg)
    exb = jnp.dot(ex, mht_ref[...], preferred_element_type=_f32)
    for k in range(D):
        out_ref[:, k * C:(k + 1) * C] = msg[k] * exb
    out_ref[:, F:FX] = exb


def _edge(xs2, qd, pne_t, conn_row, wam_t, wk, mh, mht, sw, ga_t, gb_t, sbc):
    return pl.pallas_call(
        _edge_body,
        grid=(E // BE,),
        in_specs=[
            pl.BlockSpec((BE, F), lambda i: (i, 0)),
            pl.BlockSpec((BE, C), lambda i: (i, 0)),
            pl.BlockSpec((10, BE), lambda i: (0, i)),
            pl.BlockSpec((1, BE), lambda i: (0, i)),
            pl.BlockSpec((25, 10), lambda i: (0, 0)),
            pl.BlockSpec((C, C), lambda i: (0, 0)),
            pl.BlockSpec((C, NH), lambda i: (0, 0)),
            pl.BlockSpec((NH, C), lambda i: (0, 0)),
            pl.BlockSpec((25, 25), lambda i: (0, 0)),
            pl.BlockSpec((25, 3), lambda i: (0, 0)),
            pl.BlockSpec((25, 2), lambda i: (0, 0)),
            pl.BlockSpec((25, D * F), lambda i: (0, 0)),
        ],
        out_specs=pl.BlockSpec((BE, FX), lambda i: (i, 0)),
        out_shape=jax.ShapeDtypeStruct((E, FX), _f32),
    )(xs2, qd, pne_t, conn_row, wam_t, wk, mh, mht, sw, ga_t, gb_t, sbc)


def _scatter_body(exmx_hbm, dst_hbm, zeros_hbm, agg_out, ebuf, ibuf, acc):
    cid = lax.axis_index("c")
    sid = lax.axis_index("s")
    ept = E // NSUB
    rows_per_tile = NPAD // NSUB
    for p in range(NBLK // NCORES):
        blk = cid * (NBLK // NCORES) + p
        col0 = blk * CB
        for r in range(rows_per_tile // 128):
            r0 = sid * rows_per_tile + r * 128
            pltpu.sync_copy(zeros_hbm, acc.at[pl.ds(r0, 128)])
        plsc.subcore_barrier()

        def step(j, carry):
            e0 = sid * ept + j * ECH
            pltpu.sync_copy(
                exmx_hbm.at[pl.ds(e0, ECH), pl.ds(col0, CB)], ebuf)
            pltpu.sync_copy(dst_hbm.at[pl.ds(e0, ECH)], ibuf)
            pltpu.sync_copy(ebuf, acc.at[ibuf], add=True)
            return carry

        lax.fori_loop(0, ept // ECH, step, 0)
        plsc.subcore_barrier()
        for r in range(rows_per_tile // 128):
            r0 = sid * rows_per_tile + r * 128
            pltpu.sync_copy(acc.at[pl.ds(r0, 128)], ebuf.at[pl.ds(0, 128)])
            pltpu.sync_copy(ebuf.at[pl.ds(0, 128)],
                            agg_out.at[pl.ds(r0, 128), pl.ds(col0, CB)])
        plsc.subcore_barrier()


def _scatter(exmx, dst, zeros_cb):
    mesh = plsc.VectorSubcoreMesh(core_axis_name="c", subcore_axis_name="s")
    f = pl.kernel(
        _scatter_body,
        out_type=jax.ShapeDtypeStruct((NPAD, FX), _f32),
        mesh=mesh,
        scratch_types=[
            pltpu.VMEM((ECH, CB), _f32),
            pltpu.VMEM((ECH,), jnp.int32),
            pltpu.VMEM_SHARED((NPAD, CB), _f32),
        ],
    )
    return f(exmx, dst, zeros_cb)


def _node_body(agg_ref, x_ref, res_ref, pse_ref, wsam_ref, wsch_ref, bias_ref,
               out_ref, *, residual):
    kern_s = jnp.dot(pse_ref[...], wsam_ref[...], preferred_element_type=_f32)
    wsch = wsch_ref[...]
    xws = [jnp.dot(x_ref[:, d * C:(d + 1) * C], wsch,
                   preferred_element_type=_f32) for d in range(D)]
    den = agg_ref[:, F:FX]
    r = 1.0 / (den + 1e-9)
    ys = []
    for k in range(D):
        sm = kern_s[:, k:k + 1] * xws[0]
        for d in range(1, D):
            sm = sm + kern_s[:, d * D + k:d * D + k + 1] * xws[d]
        y = agg_ref[:, k * C:(k + 1) * C] * r + sm
        if k == 0:
            y = y + bias_ref[...]
        if residual:
            y = y + res_ref[:, k * C:(k + 1) * C]
        ys.append(y)
    sps = []
    for s in range(D):
        sp = float(_BMAT[s, 0]) * ys[0]
        for k in range(1, D):
            sp = sp + float(_BMAT[s, k]) * ys[k]
        sps.append(jnp.maximum(sp, 0.0))
    for d in range(D):
        z = float(_BINV[0, d]) * sps[0]
        for s in range(1, D):
            z = z + float(_BINV[s, d]) * sps[s]
        out_ref[:, d * C:(d + 1) * C] = z


def _node(aggf, x2, res2, pse_f, wsam, wsch, bias2, residual):
    body = functools.partial(_node_body, residual=residual)
    return pl.pallas_call(
        body,
        grid=(NPAD // BN,),
        in_specs=[
            pl.BlockSpec((BN, FX), lambda i: (i, 0)),
            pl.BlockSpec((BN, F), lambda i: (i, 0)),
            pl.BlockSpec((BN, F), lambda i: (i, 0)),
            pl.BlockSpec((BN, 10), lambda i: (i, 0)),
            pl.BlockSpec((10, 25), lambda i: (0, 0)),
            pl.BlockSpec((C, C), lambda i: (0, 0)),
            pl.BlockSpec((1, C), lambda i: (0, 0)),
        ],
        out_specs=pl.BlockSpec((BN, F), lambda i: (i, 0)),
        out_shape=jax.ShapeDtypeStruct((NPAD, F), _f32),
    )(aggf, x2, res2, pse_f, wsam, wsch, bias2)


def _flatten_nodes(a):
    a2 = jnp.transpose(a, (0, 2, 1)).reshape(a.shape[0], -1)
    return jnp.pad(a2, ((0, NPAD - a.shape[0]), (0, 0)))


@jax.jit
def kernel(x, edge_index, precomp_neigh_edge, precomp_self_edge, connection,
           conv1_W_ang, conv1_W_ch, conv1_W_self_ang, conv1_W_self_ch,
           conv1_Wq, conv1_Wk, conv1_bias,
           conv2_W_ang, conv2_W_ch, conv2_W_self_ang, conv2_W_self_ch,
           conv2_Wq, conv2_Wk, conv2_bias):
    x2 = _flatten_nodes(x)
    dst = edge_index[:, 0]
    src = edge_index[:, 1]
    pne_t = precomp_neigh_edge.reshape(E, -1).T
    pse_f = jnp.pad(precomp_self_edge.reshape(N, -1), ((0, NPAD - N), (0, 0)))
    conn_row = connection.reshape(1, E)
    heads = (jnp.arange(C) // (C // NH))[:, None] == jnp.arange(NH)[None, :]
    mh = heads.astype(_f32) * (1.0 / 8.0)
    mht = heads.astype(_f32).T
    zeros_cb = jnp.zeros((128, CB), _f32)
    sw = jnp.asarray(_SW)
    ga_t = jnp.asarray(_GA.T)
    gb_t = jnp.asarray(_GB.T)
    sbc = jnp.asarray(_SBC)

    y = x2
    for wang, wch, wsang, wsch, wq, wk, bias, residual in (
        (conv1_W_ang, conv1_W_ch, conv1_W_self_ang, conv1_W_self_ch,
         conv1_Wq, conv1_Wk, conv1_bias, False),
        (conv2_W_ang, conv2_W_ch, conv2_W_self_ang, conv2_W_self_ch,
         conv2_Wq, conv2_Wk, conv2_bias, True),
    ):
        wam_t = jnp.transpose(wang, (1, 0, 2, 3)).reshape(10, 25).T
        wsam = jnp.transpose(wsang, (1, 0, 2, 3)).reshape(10, 25)
        bias2 = bias.reshape(1, C)
        xw2, qn = _prep(y, wch, wq)
        xs2, qd = _gather(src, dst, xw2, qn)
        exmx = _edge(xs2, qd, pne_t, conn_row, wam_t, wk, mh, mht,
                     sw, ga_t, gb_t, sbc)
        aggf = _scatter(exmx, dst, zeros_cb)
        y = _node(aggf, y, x2, pse_f, wsam, wsch, bias2, residual)

    out = y[:N].reshape(N, D, C)
    return jnp.transpose(out, (0, 2, 1))


# speedup vs baseline: 14.5329x; 1.0326x over previous
import functools
import numpy as np
import jax
import jax.numpy as jnp
from jax import lax
from jax.experimental import pallas as pl
from jax.experimental.pallas import tpu as pltpu
from jax.experimental.pallas import tpu_sc as plsc

N = 10000
NPAD = 10240
E = 160000
C = 128
D = 5
NH = 2
F = C * D
FX = F + C
CB = 128
NBLK = FX // CB
NCORES = 2
NSUB = 16
NW = NCORES * NSUB

BE = 1280
BN = 2048

E2 = 163840
XCH = 64
GNIT = (E2 // NW) // XCH
ECH = 160
SNIT = (E2 // NSUB) // ECH
TRASH = 10200

_f32 = jnp.float32


def _nonlin_mats(order=2, num_samples=5):
    thetas = 2.0 * np.pi * np.arange(num_samples) / num_samples
    cols = [np.ones(num_samples)]
    for m in range(1, order + 1):
        cols.append(np.cos(m * thetas))
        cols.append(np.sin(m * thetas))
    B = np.stack(cols, axis=1)
    scale = np.array([1.0] + [2.0] * (2 * order))
    Binv = (B * scale[None, :]) / num_samples
    return B.astype(np.float32), Binv.astype(np.float32)


_BMAT, _BINV = _nonlin_mats()


def _edge_consts():
    sw = np.zeros((25, 25), np.float32)
    for j in range(5):
        sw[10 + j, 5 + j] = 1.0
        sw[5 + j, 10 + j] = 1.0
        sw[20 + j, 15 + j] = 1.0
        sw[15 + j, 20 + j] = 1.0
    ga = np.zeros((3, 25), np.float32)
    ga[0, 0:5] = 1.0
    ga[1, 5:15] = 1.0
    ga[2, 15:25] = 1.0
    gb = np.zeros((2, 25), np.float32)
    gb[0, 5:10] = 1.0
    gb[0, 10:15] = -1.0
    gb[1, 15:20] = 1.0
    gb[1, 20:25] = -1.0
    sbc = np.zeros((25, 5 * 640), np.float32)
    for k in range(5):
        for d in range(5):
            sbc[d * 5 + k, k * 640 + d * 128:k * 640 + (d + 1) * 128] = 1.0
    return sw, ga, gb, sbc


_SW, _GA, _GB, _SBC = _edge_consts()


def _prep_body(x_ref, wch_ref, wq_ref, xw_ref, qn_ref):
    wch = wch_ref[...]
    for d in range(D):
        xw_ref[:, d * C:(d + 1) * C] = jnp.dot(
            x_ref[:, d * C:(d + 1) * C], wch, preferred_element_type=_f32)
    qn_ref[...] = jnp.dot(x_ref[:, 0:C], wq_ref[...], preferred_element_type=_f32)


def _prep(x2, w_ch, wq):
    return pl.pallas_call(
        _prep_body,
        grid=(NPAD // BN,),
        in_specs=[
            pl.BlockSpec((BN, F), lambda i: (i, 0)),
            pl.BlockSpec((C, C), lambda i: (0, 0)),
            pl.BlockSpec((C, C), lambda i: (0, 0)),
        ],
        out_specs=[
            pl.BlockSpec((BN, F), lambda i: (i, 0)),
            pl.BlockSpec((BN, C), lambda i: (i, 0)),
        ],
        out_shape=[
            jax.ShapeDtypeStruct((NPAD, F), _f32),
            jax.ShapeDtypeStruct((NPAD, C), _f32),
        ],
    )(x2, w_ch, wq)


def _gather_body(src2_hbm, dst2_hbm, xw_hbm, qn_hbm, xs_out, qd_out,
                 sib, dib, xb0, xb1, qb0, qb1, sx0, sx1, sq0, sq1):
    cid = lax.axis_index("c")
    sid = lax.axis_index("s")
    wid = sid * NCORES + cid
    base = wid * (E2 // NW)
    pltpu.sync_copy(src2_hbm.at[pl.ds(base, E2 // NW)], sib)
    pltpu.sync_copy(dst2_hbm.at[pl.ds(base, E2 // NW)], dib)
    xb = (xb0, xb1)
    qb = (qb0, qb1)
    sx = (sx0, sx1)
    sq = (sq0, sq1)

    def issue(c, s):
        ix = sib.at[pl.ds(c * XCH, XCH)]
        iq = dib.at[pl.ds(c * XCH, XCH)]
        pltpu.async_copy(xw_hbm.at[ix], xb[s], sx[s])
        pltpu.async_copy(qn_hbm.at[iq], qb[s], sq[s])

    def consume(c, s):
        ix = sib.at[pl.ds(c * XCH, XCH)]
        iq = dib.at[pl.ds(c * XCH, XCH)]
        pltpu.make_async_copy(xw_hbm.at[ix], xb[s], sx[s]).wait()
        pltpu.make_async_copy(qn_hbm.at[iq], qb[s], sq[s]).wait()
        e0 = base + c * XCH
        pltpu.sync_copy(xb[s], xs_out.at[pl.ds(e0, XCH)])
        pltpu.sync_copy(qb[s], qd_out.at[pl.ds(e0, XCH)])

    issue(0, 0)

    @pl.loop(0, GNIT - 2, step=2)
    def _(j):
        for b in range(2):
            c = j + b
            issue(c + 1, 1 - b)
            consume(c, b)

    issue(GNIT - 1, 1)
    consume(GNIT - 2, 0)
    consume(GNIT - 1, 1)


def _gather(src2, dst2, xw2, qn):
    mesh = plsc.VectorSubcoreMesh(core_axis_name="c", subcore_axis_name="s")
    f = pl.kernel(
        _gather_body,
        out_type=[
            jax.ShapeDtypeStruct((E2, F), _f32),
            jax.ShapeDtypeStruct((E2, C), _f32),
        ],
        mesh=mesh,
        scratch_types=[
            pltpu.VMEM((E2 // NW,), jnp.int32),
            pltpu.VMEM((E2 // NW,), jnp.int32),
            pltpu.VMEM((XCH, F), _f32),
            pltpu.VMEM((XCH, F), _f32),
            pltpu.VMEM((XCH, C), _f32),
            pltpu.VMEM((XCH, C), _f32),
            pltpu.SemaphoreType.DMA,
            pltpu.SemaphoreType.DMA,
            pltpu.SemaphoreType.DMA,
            pltpu.SemaphoreType.DMA,
        ],
    )
    return f(src2, dst2, xw2, qn)


def _edge_body(xs_ref, qd_ref, pnet_ref, conn_ref, wamt_ref, wk_ref,
               mh_ref, mht_ref, sw_ref, gat_ref, gbt_ref, sbc_ref, out_ref):
    kern_t = jnp.dot(wamt_ref[...], pnet_ref[...], preferred_element_type=_f32)
    th = conn_ref[...]
    c1 = jnp.cos(th)
    s1 = jnp.sin(th)
    c2 = c1 * c1 - s1 * s1
    s2 = 2.0 * s1 * c1
    one = jnp.ones_like(th)
    t3 = jnp.concatenate([one, c1, c2], axis=0)
    t2 = jnp.concatenate([s1, s2], axis=0)
    a = jnp.dot(gat_ref[...], t3, preferred_element_type=_f32)
    b = jnp.dot(gbt_ref[...], t2, preferred_element_type=_f32)
    kern_sw = jnp.dot(sw_ref[...], kern_t, preferred_element_type=_f32)
    cc_t = a * kern_t + b * kern_sw
    xs = [xs_ref[:, d * C:(d + 1) * C] for d in range(D)]
    msg = []
    for k in range(D):
        cb = jax.lax.dot_general(
            cc_t, sbc_ref[:, k * F:(k + 1) * F],
            (((0,), (0,)), ((), ())), preferred_element_type=_f32)
        acc = cb[:, 0:C] * xs[0]
        for d in range(1, D):
            acc = acc + cb[:, d * C:(d + 1) * C] * xs[d]
        msg.append(acc)
    katt = jnp.dot(msg[0], wk_ref[...], preferred_element_type=_f32)
    qk = qd_ref[...] * katt
    lg = jnp.dot(qk, mh_ref[...], preferred_element_type=_f32)
    ex = jnp.exp(lg)
    exb = jnp.dot(ex, mht_ref[...], preferred_element_type=_f32)
    for k in range(D):
        out_ref[:, k * C:(k + 1) * C] = msg[k] * exb
    out_ref[:, F:FX] = exb


def _edge(xs2, qd, pne_t, conn_row, wam_t, wk, mh, mht, sw, ga_t, gb_t, sbc):
    return pl.pallas_call(
        _edge_body,
        grid=(E2 // BE,),
        in_specs=[
            pl.BlockSpec((BE, F), lambda i: (i, 0)),
            pl.BlockSpec((BE, C), lambda i: (i, 0)),
            pl.BlockSpec((10, BE), lambda i: (0, i)),
            pl.BlockSpec((1, BE), lambda i: (0, i)),
            pl.BlockSpec((25, 10), lambda i: (0, 0)),
            pl.BlockSpec((C, C), lambda i: (0, 0)),
            pl.BlockSpec((C, NH), lambda i: (0, 0)),
            pl.BlockSpec((NH, C), lambda i: (0, 0)),
            pl.BlockSpec((25, 25), lambda i: (0, 0)),
            pl.BlockSpec((25, 3), lambda i: (0, 0)),
            pl.BlockSpec((25, 2), lambda i: (0, 0)),
            pl.BlockSpec((25, D * F), lambda i: (0, 0)),
        ],
        out_specs=pl.BlockSpec((BE, FX), lambda i: (i, 0)),
        out_shape=jax.ShapeDtypeStruct((E2, FX), _f32),
    )(xs2, qd, pne_t, conn_row, wam_t, wk, mh, mht, sw, ga_t, gb_t, sbc)


def _scatter_body(exmx_hbm, dst_hbm, zeros_hbm, agg_out,
                  eb0, eb1, ib0, ib1, acc, se0, se1, si0, si1):
    cid = lax.axis_index("c")
    sid = lax.axis_index("s")
    ept = E2 // NSUB
    rows_per_tile = NPAD // NSUB
    eb = (eb0, eb1)
    ib = (ib0, ib1)
    se = (se0, se1)
    si = (si0, si1)
    for p in range(NBLK // NCORES):
        blk = cid * (NBLK // NCORES) + p
        col0 = blk * CB

        def issue(c, s):
            e0 = sid * ept + c * ECH
            pltpu.async_copy(
                exmx_hbm.at[pl.ds(e0, ECH), pl.ds(col0, CB)], eb[s], se[s])
            pltpu.async_copy(dst_hbm.at[pl.ds(e0, ECH)], ib[s], si[s])

        def consume(c, s):
            e0 = sid * ept + c * ECH
            pltpu.make_async_copy(
                exmx_hbm.at[pl.ds(e0, ECH), pl.ds(col0, CB)],
                eb[s], se[s]).wait()
            pltpu.make_async_copy(
                dst_hbm.at[pl.ds(e0, ECH)], ib[s], si[s]).wait()
            pltpu.sync_copy(eb[s], acc.at[ib[s]], add=True)

        for r in range(rows_per_tile // 128):
            r0 = sid * rows_per_tile + r * 128
            pltpu.sync_copy(zeros_hbm, acc.at[pl.ds(r0, 128)])
        plsc.subcore_barrier()

        issue(0, 0)

        @pl.loop(0, SNIT - 2, step=2)
        def _(j):
            for b in range(2):
                c = j + b
                issue(c + 1, 1 - b)
                consume(c, b)

        issue(SNIT - 1, 1)
        consume(SNIT - 2, 0)
        consume(SNIT - 1, 1)
        plsc.subcore_barrier()
        for r in range(rows_per_tile // 128):
            r0 = sid * rows_per_tile + r * 128
            pltpu.sync_copy(acc.at[pl.ds(r0, 128)], eb0.at[pl.ds(0, 128)])
            pltpu.sync_copy(eb0.at[pl.ds(0, 128)],
                            agg_out.at[pl.ds(r0, 128), pl.ds(col0, CB)])
        plsc.subcore_barrier()


def _scatter(exmx, dst, zeros_cb):
    mesh = plsc.VectorSubcoreMesh(core_axis_name="c", subcore_axis_name="s")
    f = pl.kernel(
        _scatter_body,
        out_type=jax.ShapeDtypeStruct((NPAD, FX), _f32),
        mesh=mesh,
        scratch_types=[
            pltpu.VMEM((ECH, CB), _f32),
            pltpu.VMEM((ECH, CB), _f32),
            pltpu.VMEM((ECH,), jnp.int32),
            pltpu.VMEM((ECH,), jnp.int32),
            pltpu.VMEM_SHARED((NPAD, CB), _f32),
            pltpu.SemaphoreType.DMA,
            pltpu.SemaphoreType.DMA,
            pltpu.SemaphoreType.DMA,
            pltpu.SemaphoreType.DMA,
        ],
    )
    return f(exmx, dst, zeros_cb)


def _node_body(agg_ref, x_ref, res_ref, pse_ref, wsam_ref, wsch_ref, bias_ref,
               out_ref, *, residual):
    kern_s = jnp.dot(pse_ref[...], wsam_ref[...], preferred_element_type=_f32)
    wsch = wsch_ref[...]
    xws = [jnp.dot(x_ref[:, d * C:(d + 1) * C], wsch,
                   preferred_element_type=_f32) for d in range(D)]
    den = agg_ref[:, F:FX]
    r = 1.0 / (den + 1e-9)
    ys = []
    for k in range(D):
        sm = kern_s[:, k:k + 1] * xws[0]
        for d in range(1, D):
            sm = sm + kern_s[:, d * D + k:d * D + k + 1] * xws[d]
        y = agg_ref[:, k * C:(k + 1) * C] * r + sm
        if k == 0:
            y = y + bias_ref[...]
        if residual:
            y = y + res_ref[:, k * C:(k + 1) * C]
        ys.append(y)
    sps = []
    for s in range(D):
        sp = float(_BMAT[s, 0]) * ys[0]
        for k in range(1, D):
            sp = sp + float(_BMAT[s, k]) * ys[k]
        sps.append(jnp.maximum(sp, 0.0))
    for d in range(D):
        z = float(_BINV[0, d]) * sps[0]
        for s in range(1, D):
            z = z + float(_BINV[s, d]) * sps[s]
        out_ref[:, d * C:(d + 1) * C] = z


def _node(aggf, x2, res2, pse_f, wsam, wsch, bias2, residual):
    body = functools.partial(_node_body, residual=residual)
    return pl.pallas_call(
        body,
        grid=(NPAD // BN,),
        in_specs=[
            pl.BlockSpec((BN, FX), lambda i: (i, 0)),
            pl.BlockSpec((BN, F), lambda i: (i, 0)),
            pl.BlockSpec((BN, F), lambda i: (i, 0)),
            pl.BlockSpec((BN, 10), lambda i: (i, 0)),
            pl.BlockSpec((10, 25), lambda i: (0, 0)),
            pl.BlockSpec((C, C), lambda i: (0, 0)),
            pl.BlockSpec((1, C), lambda i: (0, 0)),
        ],
        out_specs=pl.BlockSpec((BN, F), lambda i: (i, 0)),
        out_shape=jax.ShapeDtypeStruct((NPAD, F), _f32),
    )(aggf, x2, res2, pse_f, wsam, wsch, bias2)


def _flatten_nodes(a):
    a2 = jnp.transpose(a, (0, 2, 1)).reshape(a.shape[0], -1)
    return jnp.pad(a2, ((0, NPAD - a.shape[0]), (0, 0)))


@jax.jit
def kernel(x, edge_index, precomp_neigh_edge, precomp_self_edge, connection,
           conv1_W_ang, conv1_W_ch, conv1_W_self_ang, conv1_W_self_ch,
           conv1_Wq, conv1_Wk, conv1_bias,
           conv2_W_ang, conv2_W_ch, conv2_W_self_ang, conv2_W_self_ch,
           conv2_Wq, conv2_Wk, conv2_bias):
    x2 = _flatten_nodes(x)
    epad = E2 - E
    dst = jnp.pad(edge_index[:, 0], (0, epad), constant_values=TRASH)
    src = jnp.pad(edge_index[:, 1], (0, epad))
    pne_t = jnp.pad(precomp_neigh_edge.reshape(E, -1).T, ((0, 0), (0, epad)))
    pse_f = jnp.pad(precomp_self_edge.reshape(N, -1), ((0, NPAD - N), (0, 0)))
    conn_row = jnp.pad(connection.reshape(1, E), ((0, 0), (0, epad)))
    heads = (jnp.arange(C) // (C // NH))[:, None] == jnp.arange(NH)[None, :]
    mh = heads.astype(_f32) * (1.0 / 8.0)
    mht = heads.astype(_f32).T
    zeros_cb = jnp.zeros((128, CB), _f32)
    sw = jnp.asarray(_SW)
    ga_t = jnp.asarray(_GA.T)
    gb_t = jnp.asarray(_GB.T)
    sbc = jnp.asarray(_SBC)

    y = x2
    for wang, wch, wsang, wsch, wq, wk, bias, residual in (
        (conv1_W_ang, conv1_W_ch, conv1_W_self_ang, conv1_W_self_ch,
         conv1_Wq, conv1_Wk, conv1_bias, False),
        (conv2_W_ang, conv2_W_ch, conv2_W_self_ang, conv2_W_self_ch,
         conv2_Wq, conv2_Wk, conv2_bias, True),
    ):
        wam_t = jnp.transpose(wang, (1, 0, 2, 3)).reshape(10, 25).T
        wsam = jnp.transpose(wsang, (1, 0, 2, 3)).reshape(10, 25)
        bias2 = bias.reshape(1, C)
        xw2, qn = _prep(y, wch, wq)
        xs2, qd = _gather(src, dst, xw2, qn)
        exmx = _edge(xs2, qd, pne_t, conn_row, wam_t, wk, mh, mht,
                     sw, ga_t, gb_t, sbc)
        aggf = _scatter(exmx, dst, zeros_cb)
        y = _node(aggf, y, x2, pse_f, wsam, wsch, bias2, residual)

    out = y[:N].reshape(N, D, C)
    return jnp.transpose(out, (0, 2, 1))
